# baseline (speedup 1.0000x reference)
import functools
import math

import jax
import jax.numpy as jnp
from jax import lax
from jax.experimental import pallas as pl
from jax.experimental.pallas import tpu as pltpu
from jax.experimental.pallas import tpu_sc as plsc

_SMOOTHING = 0.1
_CONFIDENCE = 1.0 - _SMOOTHING
_IGNORE = 0

_BATCH = 1024
_VOCAB = 100000
_EPS = _SMOOTHING / (_VOCAB - 1)
_TLOGT = (_VOCAB - 1) * _EPS * math.log(_EPS) + _CONFIDENCE * math.log(
    _CONFIDENCE
)

_K = 256
_BTC = _BATCH - _K
_ROWS = 64
_NB = _BTC // _ROWS
_GISS = 8

_NC, _NS = 2, 16
_NW = _NC * _NS
_RT = _K // _NW
_RG = _RT // 8
_CH = 4096
_NFULL = 24
_TAIL = 1696
_NBUF = 3


def _tc_body(pred_ref, tgt_ref, pred_any, tgtsc_smem, tgtsc_ref, out_ref,
             acc_ref, gbuf, gsem):
    j = pl.program_id(0)

    @pl.when(j == 0)
    def _():
        acc_ref[0] = 0.0

    @pl.when(j < _GISS)
    def _():
        per = _K // _GISS

        def issue(r, carry):
            t = tgtsc_smem[r]
            cbase = pl.multiple_of((t // 128) * 128, 128)
            rbase = _BTC + (r // 8) * 8
            pltpu.make_async_copy(
                pred_any.at[pl.ds(rbase, 8), pl.ds(cbase, 128)],
                gbuf.at[r],
                gsem,
            ).start()
            return carry

        lax.fori_loop(j * per, (j + 1) * per, issue, 0)

    x = pred_ref[...]
    tgt = tgt_ref[...]
    valid = tgt != _IGNORE
    validf = valid.astype(jnp.float32)
    rowsum = jnp.sum(x, axis=1, keepdims=True)
    col = lax.broadcasted_iota(jnp.int32, x.shape, 1)
    gathered = jnp.sum(jnp.where(col == tgt, x, 0.0), axis=1, keepdims=True)
    acc_ref[0] += jnp.sum(
        validf
        * (_TLOGT - _EPS * rowsum - (_CONFIDENCE - _EPS) * gathered)
    )

    @pl.when(j == _NB - 1)
    def _():
        def drain(r, carry):
            pltpu.make_async_copy(
                pred_any.at[pl.ds(0, 8), pl.ds(0, 128)], gbuf.at[r], gsem
            ).wait()
            return carry

        lax.fori_loop(0, _K, drain, 0)
        tsc = tgtsc_ref[...]
        vscf = (tsc != _IGNORE).astype(jnp.float32)
        lanes3 = (tsc % 128).reshape(_K, 1, 1)
        subs3 = (
            lax.broadcasted_iota(jnp.int32, (_K, 1, 1), 0) % 8
        )
        isub = lax.broadcasted_iota(jnp.int32, (_K, 8, 128), 1)
        ilane = lax.broadcasted_iota(jnp.int32, (_K, 8, 128), 2)
        sel = (isub == subs3) & (ilane == lanes3)
        g = jnp.sum(
            jnp.where(sel, gbuf[...], 0.0), axis=(1, 2)
        ).reshape(_K, 1)
        acc_ref[0] += jnp.sum(
            vscf * (_TLOGT - (_CONFIDENCE - _EPS) * g)
        )
        out_ref[0, 0] = acc_ref[0]


def _tc_part(pred, tgt_tc, tgt_sc):
    return pl.pallas_call(
        _tc_body,
        grid=(_NB,),
        in_specs=[
            pl.BlockSpec((_ROWS, _VOCAB), lambda j: (j, 0)),
            pl.BlockSpec((_ROWS, 1), lambda j: (j, 0)),
            pl.BlockSpec(memory_space=pl.ANY),
            pl.BlockSpec(memory_space=pltpu.SMEM),
            pl.BlockSpec((_K, 1), lambda j: (0, 0)),
        ],
        out_specs=pl.BlockSpec(
            (1, 1), lambda j: (0, 0), memory_space=pltpu.SMEM
        ),
        out_shape=jax.ShapeDtypeStruct((1, 1), jnp.float32),
        scratch_shapes=[
            pltpu.SMEM((1,), jnp.float32),
            pltpu.VMEM((_K, 8, 128), jnp.float32),
            pltpu.SemaphoreType.DMA,
        ],
        compiler_params=pltpu.CompilerParams(
            dimension_semantics=("arbitrary",),
            disable_bounds_checks=True,
        ),
    )(pred, tgt_tc, pred, tgt_sc.reshape(_K), tgt_sc)


def _sc_body(pred_hbm, out_hbm, b0, b1, b2, tailbuf, out_v, sem):
    bufs = (b0, b1, b2)
    wid = lax.axis_index("s") * _NC + lax.axis_index("c")
    base_row = _BTC + wid * _RT
    lane = lax.broadcasted_iota(jnp.int32, (16,), 0)
    row_sums = jnp.zeros((16,), jnp.float32)
    for g in range(_RG):
        r0 = base_row + g * 8
        for b in range(_NBUF):
            pltpu.async_copy(
                pred_hbm.at[pl.ds(r0, 8), pl.ds(b * _CH, _CH)], bufs[b], sem
            )
        accs = tuple(jnp.zeros((16,), jnp.float32) for _ in range(8))

        def group_body(k, accs, _r0=r0):
            for b in range(_NBUF):
                ci = k * _NBUF + b
                pltpu.make_async_copy(
                    pred_hbm.at[pl.ds(_r0, 8), pl.ds(0, _CH)], bufs[b], sem
                ).wait()

                def add_body(i, a, _b=b):
                    base = i * 32
                    a = tuple(
                        v + bufs[_b][r, pl.ds(base, 16)]
                        for r, v in enumerate(a)
                    )
                    return tuple(
                        v + bufs[_b][r, pl.ds(base + 16, 16)]
                        for r, v in enumerate(a)
                    )

                accs = lax.fori_loop(0, _CH // 32, add_body, accs)
                nxt = ci + _NBUF

                @pl.when(nxt < _NFULL)
                def _(_b=b, _nxt=nxt, _r0=_r0):
                    pltpu.async_copy(
                        pred_hbm.at[pl.ds(_r0, 8), pl.ds(_nxt * _CH, _CH)],
                        bufs[_b],
                        sem,
                    )

            return accs

        accs = lax.fori_loop(0, _NFULL // _NBUF, group_body, accs)
        pltpu.sync_copy(
            pred_hbm.at[pl.ds(r0, 8), pl.ds(_NFULL * _CH, _TAIL)], tailbuf
        )

        def tail_body(i, a):
            return tuple(
                v + tailbuf[r, pl.ds(i * 16, 16)] for r, v in enumerate(a)
            )

        accs = lax.fori_loop(0, _TAIL // 16, tail_body, accs)
        for r in range(8):
            s = jnp.sum(accs[r])
            row_sums = row_sums + jnp.where(lane == g * 8 + r, s, 0.0)
    out_v[...] = row_sums
    pltpu.sync_copy(
        out_v.at[pl.ds(0, _RT)], out_hbm.at[pl.ds(wid * _RT, _RT)]
    )


_sc_rowsum = functools.partial(
    pl.kernel,
    _sc_body,
    out_type=jax.ShapeDtypeStruct((_K,), jnp.float32),
    mesh=plsc.VectorSubcoreMesh(core_axis_name="c", subcore_axis_name="s"),
    compiler_params=pltpu.CompilerParams(needs_layout_passes=False),
    scratch_types=[pltpu.VMEM((8, _CH), jnp.float32)] * _NBUF
    + [
        pltpu.VMEM((8, _TAIL), jnp.float32),
        pltpu.VMEM((16,), jnp.float32),
        pltpu.SemaphoreType.DMA,
    ],
)()


def _comb_body(s_ref, rs_ref, tgt_ref, out_ref):
    rs = rs_ref[...]
    t = tgt_ref[...]
    masked = jnp.sum(jnp.where(t != _IGNORE, rs, 0.0))
    out_ref[0, 0] = (s_ref[0, 0] - _EPS * masked) / _BATCH


def _combine(s_tc, sc_sums, tgt_sc):
    return pl.pallas_call(
        _comb_body,
        in_specs=[
            pl.BlockSpec(memory_space=pltpu.SMEM),
            pl.BlockSpec((_K // 128, 128), lambda: (0, 0)),
            pl.BlockSpec((_K // 128, 128), lambda: (0, 0)),
        ],
        out_specs=pl.BlockSpec(memory_space=pltpu.SMEM),
        out_shape=jax.ShapeDtypeStruct((1, 1), jnp.float32),
    )(s_tc, sc_sums.reshape(_K // 128, 128), tgt_sc.reshape(_K // 128, 128))


def kernel(pred_logprob, target):
    tgt_tc = target[:_BTC].reshape(_BTC, 1)
    tgt_sc = target[_BTC:].reshape(_K, 1)
    s_tc = _tc_part(pred_logprob, tgt_tc, tgt_sc)
    sc_sums = _sc_rowsum(pred_logprob)
    out = _combine(s_tc, sc_sums, tgt_sc)
    return out.reshape(())
